# Initial kernel scaffold; baseline (speedup 1.0000x reference)
import jax
import jax.numpy as jnp
from jax.experimental import pallas as pl


def kernel(embs, cat_embs, premium, sentiment, mask, temporal, weekdays, hours, W1, b1, W2, b2, premium_tab, sentiment_tab, temporal_tab, weekday_tab, hour_tab):
    raise NotImplementedError("write your pallas kernel here")


# speedup vs baseline: 2.5158x; 2.5158x over previous
import functools

import jax
import jax.numpy as jnp
from jax.experimental import pallas as pl

_SELU_SCALE = 1.0507009873554805
_SELU_ALPHA = 1.6732632423543772

_OFFSETS = (0, 2, 5, 37, 44)
_TAB_ROWS = 68
_TAB_PAD = 128

_BLOCK_ROWS = 2048


def _fused_kernel(emb_ref, cat_ref, idx_ref, w1a_ref, w1b_ref, b1_ref,
                  w2_ref, b2_ref, tab_ref, out_ref):
    h = jnp.dot(emb_ref[...], w1a_ref[...], preferred_element_type=jnp.float32)
    h += jnp.dot(cat_ref[...], w1b_ref[...], preferred_element_type=jnp.float32)
    h += b1_ref[...]
    h = _SELU_SCALE * jnp.where(h > 0, h, _SELU_ALPHA * (jnp.exp(h) - 1.0))
    x = jnp.dot(h, w2_ref[...], preferred_element_type=jnp.float32)
    x += b2_ref[...]
    iota = jax.lax.broadcasted_iota(jnp.int32, (1, _TAB_PAD), 1)
    oh = jnp.zeros((emb_ref.shape[0], _TAB_PAD), dtype=jnp.float32)
    for t in range(5):
        idx_t = idx_ref[t, :][:, None]
        oh += (idx_t == iota).astype(jnp.float32)
    x += jnp.dot(oh, tab_ref[...], preferred_element_type=jnp.float32)
    out_ref[...] = x


@functools.partial(jax.jit, static_argnames=())
def kernel(embs, cat_embs, premium, sentiment, mask, temporal, weekdays, hours,
           W1, b1, W2, b2, premium_tab, sentiment_tab, temporal_tab,
           weekday_tab, hour_tab):
    B, L, ART = embs.shape
    CAT = cat_embs.shape[2]
    DIMS = W2.shape[1]
    N = B * L
    R = _BLOCK_ROWS
    grid = N // R

    embs2 = embs.reshape(N, ART)
    cat2 = cat_embs.reshape(N, CAT)

    idx = jnp.stack([
        premium.reshape(N).astype(jnp.int32) + _OFFSETS[0],
        sentiment.reshape(N).astype(jnp.int32) + _OFFSETS[1],
        temporal.reshape(N).astype(jnp.int32) + _OFFSETS[2],
        weekdays.reshape(N).astype(jnp.int32) + _OFFSETS[3],
        hours.reshape(N).astype(jnp.int32) + _OFFSETS[4],
    ])
    idx = jnp.concatenate([idx, jnp.full((3, N), _TAB_ROWS, jnp.int32)])

    tab = jnp.concatenate([premium_tab, sentiment_tab, temporal_tab,
                           weekday_tab, hour_tab,
                           jnp.zeros((_TAB_PAD - _TAB_ROWS, DIMS), jnp.float32)])

    W1a = W1[:ART]
    W1b = W1[ART:]
    b1r = b1.reshape(1, DIMS)
    b2r = b2.reshape(1, DIMS)

    out = pl.pallas_call(
        _fused_kernel,
        grid=(grid,),
        in_specs=[
            pl.BlockSpec((R, ART), lambda i: (i, 0)),
            pl.BlockSpec((R, CAT), lambda i: (i, 0)),
            pl.BlockSpec((8, R), lambda i: (0, i)),
            pl.BlockSpec((ART, DIMS), lambda i: (0, 0)),
            pl.BlockSpec((CAT, DIMS), lambda i: (0, 0)),
            pl.BlockSpec((1, DIMS), lambda i: (0, 0)),
            pl.BlockSpec((DIMS, DIMS), lambda i: (0, 0)),
            pl.BlockSpec((1, DIMS), lambda i: (0, 0)),
            pl.BlockSpec((_TAB_PAD, DIMS), lambda i: (0, 0)),
        ],
        out_specs=pl.BlockSpec((R, DIMS), lambda i: (i, 0)),
        out_shape=jax.ShapeDtypeStruct((N, DIMS), jnp.float32),
    )(embs2, cat2, idx, W1a, W1b, b1r, W2, b2r, tab)

    return (out.reshape(B, L, DIMS), mask)


# speedup vs baseline: 2.9362x; 1.1671x over previous
import jax
import jax.numpy as jnp
from jax.experimental import pallas as pl

_SELU_SCALE = 1.0507009873554805
_SELU_ALPHA = 1.6732632423543772

_OFFSETS = (0, 2, 5, 37, 44)
_TAB_ROWS = 68
_TAB_PAD = 128

_BLOCK_B = 128


def _fused_kernel(emb_ref, cat_ref, prem_ref, sent_ref, temp_ref, week_ref,
                  hour_ref, w1a_ref, w1b_ref, b1_ref, w2_ref, b2_ref, tab_ref,
                  out_ref):
    L = emb_ref.shape[1]
    iota = jax.lax.broadcasted_iota(jnp.int32, (1, _TAB_PAD), 1)
    idx_refs = (prem_ref, sent_ref, temp_ref, week_ref, hour_ref)
    for l in range(L):
        h = jnp.dot(emb_ref[:, l, :], w1a_ref[...],
                    preferred_element_type=jnp.float32)
        h += jnp.dot(cat_ref[:, l, :], w1b_ref[...],
                     preferred_element_type=jnp.float32)
        h += b1_ref[...]
        h = _SELU_SCALE * jnp.where(h > 0, h, _SELU_ALPHA * (jnp.exp(h) - 1.0))
        x = jnp.dot(h, w2_ref[...], preferred_element_type=jnp.float32)
        x += b2_ref[...]
        oh = jnp.zeros((emb_ref.shape[0], _TAB_PAD), dtype=jnp.float32)
        for t in range(5):
            oh += ((idx_refs[t][:, l:l + 1] + _OFFSETS[t]) == iota
                   ).astype(jnp.float32)
        x += jnp.dot(oh, tab_ref[...], preferred_element_type=jnp.float32)
        out_ref[:, l, :] = x


def kernel(embs, cat_embs, premium, sentiment, mask, temporal, weekdays, hours,
           W1, b1, W2, b2, premium_tab, sentiment_tab, temporal_tab,
           weekday_tab, hour_tab):
    B, L, ART = embs.shape
    CAT = cat_embs.shape[2]
    DIMS = W2.shape[1]
    RB = _BLOCK_B
    grid = B // RB

    tab = jnp.concatenate([premium_tab, sentiment_tab, temporal_tab,
                           weekday_tab, hour_tab,
                           jnp.zeros((_TAB_PAD - _TAB_ROWS, DIMS), jnp.float32)])
    W1a = W1[:ART]
    W1b = W1[ART:]
    b1r = b1.reshape(1, DIMS)
    b2r = b2.reshape(1, DIMS)

    idx_spec = pl.BlockSpec((RB, L), lambda i: (i, 0))
    out = pl.pallas_call(
        _fused_kernel,
        grid=(grid,),
        in_specs=[
            pl.BlockSpec((RB, L, ART), lambda i: (i, 0, 0)),
            pl.BlockSpec((RB, L, CAT), lambda i: (i, 0, 0)),
            idx_spec, idx_spec, idx_spec, idx_spec, idx_spec,
            pl.BlockSpec((ART, DIMS), lambda i: (0, 0)),
            pl.BlockSpec((CAT, DIMS), lambda i: (0, 0)),
            pl.BlockSpec((1, DIMS), lambda i: (0, 0)),
            pl.BlockSpec((DIMS, DIMS), lambda i: (0, 0)),
            pl.BlockSpec((1, DIMS), lambda i: (0, 0)),
            pl.BlockSpec((_TAB_PAD, DIMS), lambda i: (0, 0)),
        ],
        out_specs=pl.BlockSpec((RB, L, DIMS), lambda i: (i, 0, 0)),
        out_shape=jax.ShapeDtypeStruct((B, L, DIMS), jnp.float32),
    )(embs, cat_embs, premium.astype(jnp.int32), sentiment.astype(jnp.int32),
      temporal.astype(jnp.int32), weekdays.astype(jnp.int32),
      hours.astype(jnp.int32), W1a, W1b, b1r, W2, b2r, tab)

    return (out, mask)
